# baseline (speedup 1.0000x reference)
import jax
import jax.numpy as jnp
from jax.experimental import pallas as pl
from jax.experimental.pallas import tpu as pltpu

DM = 768
DI = 1536
NS = 16
RK = 48
L = 1024
BA = 2
EPS = 1e-5
DT = 128
NT = DI // DT
SB = 128


def _ln_inproj_body(x_ref, g_ref, b_ref, w_ref, o_ref):
    xv = x_ref[...]
    mu = jnp.mean(xv, axis=-1, keepdims=True)
    d = xv - mu
    var = jnp.mean(d * d, axis=-1, keepdims=True)
    xn = d * jax.lax.rsqrt(var + EPS) * g_ref[...] + b_ref[...]
    o_ref[...] = jnp.dot(xn, w_ref[...], preferred_element_type=jnp.float32)


def _conv_core(reverse, xz_ref, cw_ref, cb_ref, xp_ref, dtw_ref, dtb_ref,
               de_ref, xc_ref, bc_ref):
        xs = xz_ref[0]
        acc = cb_ref[0] + cw_ref[0, 0:1, :] * xs
        for j in (1, 2, 3):
            w = cw_ref[0, j:j + 1, :]
            if reverse:
                c1 = xs[:j * SB]
                parts = []
                for m in range(j):
                    parts.append(jnp.concatenate(
                        [c1[m * SB + 1:(m + 1) * SB],
                         jnp.zeros((1, DI), jnp.float32)], axis=0))
                seg = jnp.concatenate([xs[j * SB:]] + parts, axis=0)
            else:
                c1 = xs[(8 - j) * SB:]
                parts = []
                for m in range(j):
                    parts.append(jnp.concatenate(
                        [jnp.zeros((1, DI), jnp.float32),
                         c1[m * SB:(m + 1) * SB - 1]], axis=0))
                seg = jnp.concatenate(parts + [xs[:(8 - j) * SB]], axis=0)
            acc = acc + w * seg
        xc = acc * jax.nn.sigmoid(acc)
        proj = jnp.dot(xc, xp_ref[0], preferred_element_type=jnp.float32)
        dlin = jnp.dot(proj[:, 0:RK], dtw_ref[0],
                       preferred_element_type=jnp.float32) + dtb_ref[0]
        de_ref[0] = jax.nn.softplus(dlin)
        xc_ref[0] = xc
        bc_ref[0] = proj[:, RK:RK + 2 * NS]


def _conv_body(xz_ref, cw_ref, cb_ref, xp_ref, dtw_ref, dtb_ref,
               de_ref, xc_ref, bc_ref):
    d = pl.program_id(0)

    @pl.when(d == 0)
    def _():
        _conv_core(False, xz_ref, cw_ref, cb_ref, xp_ref, dtw_ref, dtb_ref,
                   de_ref, xc_ref, bc_ref)

    @pl.when(d == 1)
    def _():
        _conv_core(True, xz_ref, cw_ref, cb_ref, xp_ref, dtw_ref, dtb_ref,
                   de_ref, xc_ref, bc_ref)


def _scan_core(reverse, de_ref, xc_ref, bc_ref, z_ref, a_ref, dvec_ref, y_ref):
        bc = bc_ref[0]
        dxv = de_ref[0] * xc_ref[0]
        for n in range(NS):
            a_row = a_ref[0, n:n + 1, :]
            A = jnp.exp(de_ref[0] * a_row)
            B = dxv * bc[:, n:n + 1]
            for k in (1, 2, 4):
                sh = k * SB
                if reverse:
                    B = jnp.concatenate(
                        [B[:L - sh] + A[:L - sh] * B[sh:], B[L - sh:]], axis=0)
                    A = jnp.concatenate(
                        [A[:L - sh] * A[sh:], A[L - sh:]], axis=0)
                else:
                    B = jnp.concatenate(
                        [B[:sh], B[sh:] + A[sh:] * B[:L - sh]], axis=0)
                    A = jnp.concatenate(
                        [A[:sh], A[sh:] * A[:L - sh]], axis=0)
            if reverse:
                As, Bs = A[0:SB], B[0:SB]
            else:
                As, Bs = A[L - SB:], B[L - SB:]
            for i, k in enumerate((1, 2, 4, 8, 16, 32, 64)):
                if reverse:
                    Bs_new = jnp.concatenate(
                        [Bs[:SB - k] + As[:SB - k] * Bs[k:], Bs[SB - k:]], axis=0)
                    if i < 6:
                        As = jnp.concatenate(
                            [As[:SB - k] * As[k:], As[SB - k:]], axis=0)
                else:
                    Bs_new = jnp.concatenate(
                        [Bs[:k], Bs[k:] + As[k:] * Bs[:SB - k]], axis=0)
                    if i < 6:
                        As = jnp.concatenate(
                            [As[:k], As[k:] * As[:SB - k]], axis=0)
                Bs = Bs_new
            c_col = bc[:, NS + n:NS + n + 1]
            if reverse:
                carry = jnp.concatenate(
                    [Bs[1:], jnp.zeros((1, DT), jnp.float32)], axis=0)
                carry7 = jnp.concatenate([carry] * 7, axis=0)
                h = jnp.concatenate(
                    [Bs, B[SB:] + A[SB:] * carry7], axis=0)
            else:
                carry = jnp.concatenate(
                    [jnp.zeros((1, DT), jnp.float32), Bs[:-1]], axis=0)
                carry7 = jnp.concatenate([carry] * 7, axis=0)
                h = jnp.concatenate(
                    [B[:L - SB] + A[:L - SB] * carry7, Bs], axis=0)
            contrib = h * c_col
            if n == 0:
                y_ref[0] = contrib
            else:
                y_ref[0] = y_ref[0] + contrib
        zv = z_ref[0]
        y_ref[0] = ((y_ref[0] + xc_ref[0] * dvec_ref[0])
                    * (zv * jax.nn.sigmoid(zv)))


def _scan_body(de_ref, xc_ref, bc_ref, z_ref, a_ref, dvec_ref, y_ref):
    d = pl.program_id(0)

    @pl.when(d == 0)
    def _():
        _scan_core(False, de_ref, xc_ref, bc_ref, z_ref, a_ref, dvec_ref, y_ref)

    @pl.when(d == 1)
    def _():
        _scan_core(True, de_ref, xc_ref, bc_ref, z_ref, a_ref, dvec_ref, y_ref)


def _outproj_body(y_ref, w_ref, x_ref, o_ref):
    o_ref[...] = x_ref[...] + jnp.dot(y_ref[...], w_ref[...],
                                      preferred_element_type=jnp.float32)


def kernel(x, gamma, beta,
           f_in_w, f_conv_w, f_conv_b, f_xproj_w, f_dt_w, f_dt_b, f_A_log, f_D, f_out_w,
           b_in_w, b_conv_w, b_conv_b, b_xproj_w, b_dt_w, b_dt_b, b_A_log, b_D, b_out_w):
    f32 = jnp.float32
    xr = x.reshape(BA, L // 8, 8, DM).transpose(0, 2, 1, 3).reshape(BA * L, DM)
    w_in = jnp.concatenate([f_in_w, b_in_w], axis=0).T
    cw_f = f_conv_w[:, ::-1].T
    cw_b = b_conv_w[:, ::-1].T
    cb_f = f_conv_b.reshape(1, 1, DI)
    cb_b = b_conv_b.reshape(1, 1, DI)
    a_f = -jnp.exp(f_A_log.T).reshape(1, NS, DI)
    a_b = -jnp.exp(b_A_log.T).reshape(1, NS, DI)
    d_f = f_D.reshape(1, 1, DI)
    d_b = b_D.reshape(1, 1, DI)
    w_out_f = f_out_w.T
    w_out_b = b_out_w.T

    xz = pl.pallas_call(
        _ln_inproj_body,
        grid=(4, 8),
        in_specs=[
            pl.BlockSpec((256, DM), lambda j, i: (i, 0)),
            pl.BlockSpec((1, DM), lambda j, i: (0, 0)),
            pl.BlockSpec((1, DM), lambda j, i: (0, 0)),
            pl.BlockSpec((DM, DI), lambda j, i: (0, j)),
        ],
        out_specs=pl.BlockSpec((256, DI), lambda j, i: (i, j)),
        out_shape=jax.ShapeDtypeStruct((BA * L, 4 * DI), f32),
        compiler_params=pltpu.CompilerParams(
            dimension_semantics=("parallel", "arbitrary"),
            vmem_limit_bytes=48 * 1024 * 1024),
        name="ln_inproj",
    )(xr, gamma.reshape(1, DM), beta.reshape(1, DM), w_in)
    xz3 = xz.reshape(BA, L, 4 * DI)

    cw_s = jnp.stack([cw_f, cw_b])
    cb_s = jnp.concatenate([cb_f, cb_b], axis=0)
    xp_s = jnp.stack([f_xproj_w.T, b_xproj_w.T])
    dtw_s = jnp.stack([f_dt_w.T, b_dt_w.T])
    dtb_s = jnp.stack([f_dt_b.reshape(1, DI), b_dt_b.reshape(1, DI)])
    a_s = jnp.concatenate([a_f, a_b], axis=0)
    d_s = jnp.concatenate([d_f, d_b], axis=0)

    de_a, xc_a, bc_a = pl.pallas_call(
        _conv_body,
        grid=(2, BA),
        in_specs=[
            pl.BlockSpec((1, L, DI), lambda d, b: (b, 0, 2 * d)),
            pl.BlockSpec((1, 4, DI), lambda d, b: (d, 0, 0)),
            pl.BlockSpec((1, 1, DI), lambda d, b: (d, 0, 0)),
            pl.BlockSpec((1, DI, RK + 2 * NS), lambda d, b: (d, 0, 0)),
            pl.BlockSpec((1, RK, DI), lambda d, b: (d, 0, 0)),
            pl.BlockSpec((1, 1, DI), lambda d, b: (d, 0, 0)),
        ],
        out_specs=[
            pl.BlockSpec((1, L, DI), lambda d, b: (2 * d + b, 0, 0)),
            pl.BlockSpec((1, L, DI), lambda d, b: (2 * d + b, 0, 0)),
            pl.BlockSpec((1, L, 2 * NS), lambda d, b: (2 * d + b, 0, 0)),
        ],
        out_shape=[
            jax.ShapeDtypeStruct((2 * BA, L, DI), f32),
            jax.ShapeDtypeStruct((2 * BA, L, DI), f32),
            jax.ShapeDtypeStruct((2 * BA, L, 2 * NS), f32),
        ],
        compiler_params=pltpu.CompilerParams(
            dimension_semantics=("parallel", "arbitrary"),
            vmem_limit_bytes=52 * 1024 * 1024),
        name="convproj",
    )(xz3, cw_s, cb_s, xp_s, dtw_s, dtb_s.reshape(2, 1, DI))

    yg = pl.pallas_call(
        _scan_body,
        grid=(2, BA, NT),
        in_specs=[
            pl.BlockSpec((1, L, DT), lambda d, b, j: (2 * d + b, 0, j)),
            pl.BlockSpec((1, L, DT), lambda d, b, j: (2 * d + b, 0, j)),
            pl.BlockSpec((1, L, 2 * NS), lambda d, b, j: (2 * d + b, 0, 0)),
            pl.BlockSpec((1, L, DT), lambda d, b, j: (b, 0, 2 * NT * d + NT + j)),
            pl.BlockSpec((1, NS, DT), lambda d, b, j: (d, 0, j)),
            pl.BlockSpec((1, 1, DT), lambda d, b, j: (d, 0, j)),
        ],
        out_specs=pl.BlockSpec((1, L, DT), lambda d, b, j: (b, 0, d * NT + j)),
        out_shape=jax.ShapeDtypeStruct((BA, L, 2 * DI), f32),
        compiler_params=pltpu.CompilerParams(
            dimension_semantics=("parallel", "parallel", "arbitrary"),
            vmem_limit_bytes=48 * 1024 * 1024),
        name="scan",
    )(de_a, xc_a, bc_a, xz3, a_s, d_s)
    ygr = yg.reshape(BA * L, 2 * DI)

    w_out = jnp.concatenate([f_out_w, b_out_w], axis=1).T
    out = pl.pallas_call(
        _outproj_body,
        grid=(8,),
        in_specs=[
            pl.BlockSpec((256, 2 * DI), lambda i: (i, 0)),
            pl.BlockSpec((2 * DI, DM), lambda i: (0, 0)),
            pl.BlockSpec((256, DM), lambda i: (i, 0)),
        ],
        out_specs=pl.BlockSpec((256, DM), lambda i: (i, 0)),
        out_shape=jax.ShapeDtypeStruct((BA * L, DM), f32),
        compiler_params=pltpu.CompilerParams(
            dimension_semantics=("parallel",),
            vmem_limit_bytes=48 * 1024 * 1024),
        name="outproj",
    )(ygr, w_out, xr)
    return (out.reshape(BA, 8, L // 8, DM).transpose(0, 2, 1, 3)
            .reshape(BA, L, DM))


# speedup vs baseline: 1.0804x; 1.0804x over previous
import functools

import jax
import jax.numpy as jnp
from jax.experimental import pallas as pl
from jax.experimental.pallas import tpu as pltpu

DM = 768
DI = 1536
NS = 16
RK = 48
L = 1024
BA = 2
EPS = 1e-5
DT = 128
NT = DI // DT
SB = 128


def _ln_inproj_body(x_ref, g_ref, b_ref, w_ref, o_ref):
    xv = x_ref[...]
    mu = jnp.mean(xv, axis=-1, keepdims=True)
    d = xv - mu
    var = jnp.mean(d * d, axis=-1, keepdims=True)
    xn = d * jax.lax.rsqrt(var + EPS) * g_ref[...] + b_ref[...]
    o_ref[...] = jnp.dot(xn, w_ref[...], preferred_element_type=jnp.float32)


def _conv_core(reverse, xz_ref, cw_ref, cb_ref, xp_ref, dtw_ref, dtb_ref,
               de_ref, xc_ref, bc_ref):
        xs = xz_ref[0]
        acc = cb_ref[0] + cw_ref[0, 0:1, :] * xs
        for j in (1, 2, 3):
            w = cw_ref[0, j:j + 1, :]
            if reverse:
                c1 = xs[:j * SB]
                parts = []
                for m in range(j):
                    parts.append(jnp.concatenate(
                        [c1[m * SB + 1:(m + 1) * SB],
                         jnp.zeros((1, DI), jnp.float32)], axis=0))
                seg = jnp.concatenate([xs[j * SB:]] + parts, axis=0)
            else:
                c1 = xs[(8 - j) * SB:]
                parts = []
                for m in range(j):
                    parts.append(jnp.concatenate(
                        [jnp.zeros((1, DI), jnp.float32),
                         c1[m * SB:(m + 1) * SB - 1]], axis=0))
                seg = jnp.concatenate(parts + [xs[:(8 - j) * SB]], axis=0)
            acc = acc + w * seg
        xc = acc * jax.nn.sigmoid(acc)
        proj = jnp.dot(xc, xp_ref[0], preferred_element_type=jnp.float32)
        dlin = jnp.dot(proj[:, 0:RK], dtw_ref[0],
                       preferred_element_type=jnp.float32) + dtb_ref[0]
        de_ref[0] = jax.nn.softplus(dlin)
        xc_ref[0] = xc
        bc_ref[0] = proj[:, RK:RK + 2 * NS]


def _conv_body(xz_ref, cw_ref, cb_ref, xp_ref, dtw_ref, dtb_ref,
               de_ref, xc_ref, bc_ref):
    d = pl.program_id(0)

    @pl.when(d == 0)
    def _():
        _conv_core(False, xz_ref, cw_ref, cb_ref, xp_ref, dtw_ref, dtb_ref,
                   de_ref, xc_ref, bc_ref)

    @pl.when(d == 1)
    def _():
        _conv_core(True, xz_ref, cw_ref, cb_ref, xp_ref, dtw_ref, dtb_ref,
                   de_ref, xc_ref, bc_ref)


def _scan_core(reverse, de_ref, xc_ref, bc_ref, z_ref, a_ref, dvec_ref, y_ref):
        bc = bc_ref[0]
        dxv = de_ref[0] * xc_ref[0]
        for n in range(NS):
            a_row = a_ref[0, n:n + 1, :]
            A = jnp.exp(de_ref[0] * a_row)
            B = dxv * bc[:, n:n + 1]
            for k in (1, 2, 4):
                sh = k * SB
                if reverse:
                    B = jnp.concatenate(
                        [B[:L - sh] + A[:L - sh] * B[sh:], B[L - sh:]], axis=0)
                    A = jnp.concatenate(
                        [A[:L - sh] * A[sh:], A[L - sh:]], axis=0)
                else:
                    B = jnp.concatenate(
                        [B[:sh], B[sh:] + A[sh:] * B[:L - sh]], axis=0)
                    A = jnp.concatenate(
                        [A[:sh], A[sh:] * A[:L - sh]], axis=0)
            if reverse:
                As, Bs = A[0:SB], B[0:SB]
            else:
                As, Bs = A[L - SB:], B[L - SB:]
            for i, k in enumerate((1, 2, 4, 8, 16, 32, 64)):
                if reverse:
                    Bs_new = jnp.concatenate(
                        [Bs[:SB - k] + As[:SB - k] * Bs[k:], Bs[SB - k:]], axis=0)
                    if i < 6:
                        As = jnp.concatenate(
                            [As[:SB - k] * As[k:], As[SB - k:]], axis=0)
                else:
                    Bs_new = jnp.concatenate(
                        [Bs[:k], Bs[k:] + As[k:] * Bs[:SB - k]], axis=0)
                    if i < 6:
                        As = jnp.concatenate(
                            [As[:k], As[k:] * As[:SB - k]], axis=0)
                Bs = Bs_new
            c_col = bc[:, NS + n:NS + n + 1]
            if reverse:
                carry = jnp.concatenate(
                    [Bs[1:], jnp.zeros((1, DT), jnp.float32)], axis=0)
                carry7 = jnp.concatenate([carry] * 7, axis=0)
                h = jnp.concatenate(
                    [Bs, B[SB:] + A[SB:] * carry7], axis=0)
            else:
                carry = jnp.concatenate(
                    [jnp.zeros((1, DT), jnp.float32), Bs[:-1]], axis=0)
                carry7 = jnp.concatenate([carry] * 7, axis=0)
                h = jnp.concatenate(
                    [B[:L - SB] + A[:L - SB] * carry7, Bs], axis=0)
            contrib = h * c_col
            if n == 0:
                y_ref[0] = contrib
            else:
                y_ref[0] = y_ref[0] + contrib
        zv = z_ref[0]
        y_ref[0] = ((y_ref[0] + xc_ref[0] * dvec_ref[0])
                    * (zv * jax.nn.sigmoid(zv)))


def _scan_body(de_ref, xc_ref, bc_ref, z_ref, a_ref, dvec_ref, y_ref):
    d = pl.program_id(0)

    @pl.when(d == 0)
    def _():
        _scan_core(False, de_ref, xc_ref, bc_ref, z_ref, a_ref, dvec_ref, y_ref)

    @pl.when(d == 1)
    def _():
        _scan_core(True, de_ref, xc_ref, bc_ref, z_ref, a_ref, dvec_ref, y_ref)


def _outproj_body(yf_ref, yb_ref, wf_ref, wb_ref, x_ref, o_ref):
    o_ref[...] = (x_ref[...]
                  + jnp.dot(yf_ref[...], wf_ref[...],
                            preferred_element_type=jnp.float32)
                  + jnp.dot(yb_ref[...], wb_ref[...],
                            preferred_element_type=jnp.float32))


def kernel(x, gamma, beta,
           f_in_w, f_conv_w, f_conv_b, f_xproj_w, f_dt_w, f_dt_b, f_A_log, f_D, f_out_w,
           b_in_w, b_conv_w, b_conv_b, b_xproj_w, b_dt_w, b_dt_b, b_A_log, b_D, b_out_w):
    f32 = jnp.float32
    xr = x.reshape(BA, L // 8, 8, DM).transpose(0, 2, 1, 3).reshape(BA * L, DM)
    w_in = jnp.concatenate([f_in_w, b_in_w], axis=0).T
    cw_f = f_conv_w[:, ::-1].T
    cw_b = b_conv_w[:, ::-1].T
    cb_f = f_conv_b.reshape(1, 1, DI)
    cb_b = b_conv_b.reshape(1, 1, DI)
    a_f = -jnp.exp(f_A_log.T).reshape(1, NS, DI)
    a_b = -jnp.exp(b_A_log.T).reshape(1, NS, DI)
    d_f = f_D.reshape(1, 1, DI)
    d_b = b_D.reshape(1, 1, DI)
    w_out_f = f_out_w.T
    w_out_b = b_out_w.T

    xz = pl.pallas_call(
        _ln_inproj_body,
        grid=(4, 8),
        in_specs=[
            pl.BlockSpec((256, DM), lambda j, i: (i, 0)),
            pl.BlockSpec((1, DM), lambda j, i: (0, 0)),
            pl.BlockSpec((1, DM), lambda j, i: (0, 0)),
            pl.BlockSpec((DM, DI), lambda j, i: (0, j)),
        ],
        out_specs=pl.BlockSpec((256, DI), lambda j, i: (i, j)),
        out_shape=jax.ShapeDtypeStruct((BA * L, 4 * DI), f32),
        compiler_params=pltpu.CompilerParams(
            dimension_semantics=("parallel", "arbitrary"),
            vmem_limit_bytes=48 * 1024 * 1024),
        name="ln_inproj",
    )(xr, gamma.reshape(1, DM), beta.reshape(1, DM), w_in)
    xz3 = xz.reshape(BA, L, 4 * DI)

    def conv_call(reverse):
        dirn = 1 if reverse else 0
        cw = cw_b if reverse else cw_f
        cb = cb_b if reverse else cb_f
        xp = b_xproj_w if reverse else f_xproj_w
        dtw = b_dt_w if reverse else f_dt_w
        dtb = b_dt_b if reverse else f_dt_b
        return pl.pallas_call(
            functools.partial(_conv_core, reverse),
            grid=(BA,),
            in_specs=[
                pl.BlockSpec((1, L, DI), lambda b, d=dirn: (b, 0, 2 * d)),
                pl.BlockSpec((1, 4, DI), lambda b: (0, 0, 0)),
                pl.BlockSpec((1, 1, DI), lambda b: (0, 0, 0)),
                pl.BlockSpec((1, DI, RK + 2 * NS), lambda b: (0, 0, 0)),
                pl.BlockSpec((1, RK, DI), lambda b: (0, 0, 0)),
                pl.BlockSpec((1, 1, DI), lambda b: (0, 0, 0)),
            ],
            out_specs=[
                pl.BlockSpec((1, L, DI), lambda b: (b, 0, 0)),
                pl.BlockSpec((1, L, DI), lambda b: (b, 0, 0)),
                pl.BlockSpec((1, L, 2 * NS), lambda b: (b, 0, 0)),
            ],
            out_shape=[
                jax.ShapeDtypeStruct((BA, L, DI), f32),
                jax.ShapeDtypeStruct((BA, L, DI), f32),
                jax.ShapeDtypeStruct((BA, L, 2 * NS), f32),
            ],
            compiler_params=pltpu.CompilerParams(
                dimension_semantics=("parallel",),
                vmem_limit_bytes=52 * 1024 * 1024),
            name="conv_bwd" if reverse else "conv_fwd",
        )(xz3, cw.reshape(1, 4, DI), cb, xp.T.reshape(1, DI, RK + 2 * NS),
          dtw.T.reshape(1, RK, DI), dtb.reshape(1, 1, DI))

    de_f, xc_f, bc_f = conv_call(False)
    de_b, xc_b, bc_b = conv_call(True)

    def scan_call(reverse, de_a, xc_a, bc_a):
        dirn = 1 if reverse else 0
        zoff = dirn * 2 * NT + NT
        av = a_b if reverse else a_f
        dvec = d_b if reverse else d_f
        return pl.pallas_call(
            functools.partial(_scan_core, reverse),
            grid=(BA, NT),
            in_specs=[
                pl.BlockSpec((1, L, DT), lambda b, j: (b, 0, j)),
                pl.BlockSpec((1, L, DT), lambda b, j: (b, 0, j)),
                pl.BlockSpec((1, L, 2 * NS), lambda b, j: (b, 0, 0)),
                pl.BlockSpec((1, L, DT), lambda b, j, zo=zoff: (b, 0, zo + j)),
                pl.BlockSpec((1, NS, DT), lambda b, j: (0, 0, j)),
                pl.BlockSpec((1, 1, DT), lambda b, j: (0, 0, j)),
            ],
            out_specs=pl.BlockSpec((1, L, DT), lambda b, j: (b, 0, j)),
            out_shape=jax.ShapeDtypeStruct((BA, L, DI), f32),
            compiler_params=pltpu.CompilerParams(
                dimension_semantics=("parallel", "arbitrary"),
                vmem_limit_bytes=48 * 1024 * 1024),
            name="scan_bwd" if reverse else "scan_fwd",
        )(de_a, xc_a, bc_a, xz3, av, dvec)

    yg_f = scan_call(False, de_f, xc_f, bc_f).reshape(BA * L, DI)
    yg_b = scan_call(True, de_b, xc_b, bc_b).reshape(BA * L, DI)

    out = pl.pallas_call(
        _outproj_body,
        grid=(8,),
        in_specs=[
            pl.BlockSpec((256, DI), lambda i: (i, 0)),
            pl.BlockSpec((256, DI), lambda i: (i, 0)),
            pl.BlockSpec((DI, DM), lambda i: (0, 0)),
            pl.BlockSpec((DI, DM), lambda i: (0, 0)),
            pl.BlockSpec((256, DM), lambda i: (i, 0)),
        ],
        out_specs=pl.BlockSpec((256, DM), lambda i: (i, 0)),
        out_shape=jax.ShapeDtypeStruct((BA * L, DM), f32),
        compiler_params=pltpu.CompilerParams(
            dimension_semantics=("parallel",),
            vmem_limit_bytes=48 * 1024 * 1024),
        name="outproj",
    )(yg_f, yg_b, w_out_f, w_out_b, xr)
    return (out.reshape(BA, 8, L // 8, DM).transpose(0, 2, 1, 3)
            .reshape(BA, L, DM))


# speedup vs baseline: 1.0924x; 1.0111x over previous
import functools

import jax
import jax.numpy as jnp
from jax.experimental import pallas as pl
from jax.experimental.pallas import tpu as pltpu

DM = 768
DI = 1536
NS = 16
RK = 48
L = 1024
BA = 2
EPS = 1e-5
DT = 128
NT = DI // DT
SB = 128


def _ln_inproj_body(x_ref, g_ref, b_ref, w_ref, o_ref):
    xv = x_ref[...]
    mu = jnp.mean(xv, axis=-1, keepdims=True)
    d = xv - mu
    var = jnp.mean(d * d, axis=-1, keepdims=True)
    xn = d * jax.lax.rsqrt(var + EPS) * g_ref[...] + b_ref[...]
    o_ref[...] = jnp.dot(xn.astype(jnp.bfloat16), w_ref[...],
                         preferred_element_type=jnp.float32)


def _conv_core(reverse, xz_ref, cw_ref, cb_ref, xp_ref, dtw_ref, dtb_ref,
               de_ref, xc_ref, bc_ref):
        xs = xz_ref[0]
        acc = cb_ref[0] + cw_ref[0, 0:1, :] * xs
        for j in (1, 2, 3):
            w = cw_ref[0, j:j + 1, :]
            if reverse:
                c1 = xs[:j * SB]
                parts = []
                for m in range(j):
                    parts.append(jnp.concatenate(
                        [c1[m * SB + 1:(m + 1) * SB],
                         jnp.zeros((1, DI), jnp.float32)], axis=0))
                seg = jnp.concatenate([xs[j * SB:]] + parts, axis=0)
            else:
                c1 = xs[(8 - j) * SB:]
                parts = []
                for m in range(j):
                    parts.append(jnp.concatenate(
                        [jnp.zeros((1, DI), jnp.float32),
                         c1[m * SB:(m + 1) * SB - 1]], axis=0))
                seg = jnp.concatenate(parts + [xs[:(8 - j) * SB]], axis=0)
            acc = acc + w * seg
        xc = acc * jax.nn.sigmoid(acc)
        proj = jnp.dot(xc, xp_ref[0], preferred_element_type=jnp.float32)
        dlin = jnp.dot(proj[:, 0:RK], dtw_ref[0],
                       preferred_element_type=jnp.float32) + dtb_ref[0]
        de_ref[0] = jax.nn.softplus(dlin)
        xc_ref[0] = xc
        bc_ref[0] = proj[:, RK:RK + 2 * NS]


def _conv_body(xz_ref, cw_ref, cb_ref, xp_ref, dtw_ref, dtb_ref,
               de_ref, xc_ref, bc_ref):
    d = pl.program_id(0)

    @pl.when(d == 0)
    def _():
        _conv_core(False, xz_ref, cw_ref, cb_ref, xp_ref, dtw_ref, dtb_ref,
                   de_ref, xc_ref, bc_ref)

    @pl.when(d == 1)
    def _():
        _conv_core(True, xz_ref, cw_ref, cb_ref, xp_ref, dtw_ref, dtb_ref,
                   de_ref, xc_ref, bc_ref)


def _scan_core(reverse, de_ref, xc_ref, bc_ref, z_ref, a_ref, dvec_ref, y_ref):
        bc = bc_ref[0]
        dxv = de_ref[0] * xc_ref[0]
        for n in range(NS):
            a_row = a_ref[0, n:n + 1, :]
            A = jnp.exp(de_ref[0] * a_row)
            B = dxv * bc[:, n:n + 1]
            for k in (1, 2, 4):
                sh = k * SB
                if reverse:
                    B = jnp.concatenate(
                        [B[:L - sh] + A[:L - sh] * B[sh:], B[L - sh:]], axis=0)
                    A = jnp.concatenate(
                        [A[:L - sh] * A[sh:], A[L - sh:]], axis=0)
                else:
                    B = jnp.concatenate(
                        [B[:sh], B[sh:] + A[sh:] * B[:L - sh]], axis=0)
                    A = jnp.concatenate(
                        [A[:sh], A[sh:] * A[:L - sh]], axis=0)
            if reverse:
                As, Bs = A[0:SB], B[0:SB]
            else:
                As, Bs = A[L - SB:], B[L - SB:]
            for i, k in enumerate((1, 2, 4, 8, 16, 32, 64)):
                if reverse:
                    Bs_new = jnp.concatenate(
                        [Bs[:SB - k] + As[:SB - k] * Bs[k:], Bs[SB - k:]], axis=0)
                    if i < 6:
                        As = jnp.concatenate(
                            [As[:SB - k] * As[k:], As[SB - k:]], axis=0)
                else:
                    Bs_new = jnp.concatenate(
                        [Bs[:k], Bs[k:] + As[k:] * Bs[:SB - k]], axis=0)
                    if i < 6:
                        As = jnp.concatenate(
                            [As[:k], As[k:] * As[:SB - k]], axis=0)
                Bs = Bs_new
            c_col = bc[:, NS + n:NS + n + 1]
            if reverse:
                carry = jnp.concatenate(
                    [Bs[1:], jnp.zeros((1, DT), jnp.float32)], axis=0)
                carry7 = jnp.concatenate([carry] * 7, axis=0)
                h = jnp.concatenate(
                    [Bs, B[SB:] + A[SB:] * carry7], axis=0)
            else:
                carry = jnp.concatenate(
                    [jnp.zeros((1, DT), jnp.float32), Bs[:-1]], axis=0)
                carry7 = jnp.concatenate([carry] * 7, axis=0)
                h = jnp.concatenate(
                    [B[:L - SB] + A[:L - SB] * carry7, Bs], axis=0)
            contrib = h * c_col
            if n == 0:
                y_ref[0] = contrib
            else:
                y_ref[0] = y_ref[0] + contrib
        zv = z_ref[0]
        y_ref[0] = ((y_ref[0] + xc_ref[0] * dvec_ref[0])
                    * (zv * jax.nn.sigmoid(zv)))


def _scan_body(de_ref, xc_ref, bc_ref, z_ref, a_ref, dvec_ref, y_ref):
    d = pl.program_id(0)

    @pl.when(d == 0)
    def _():
        _scan_core(False, de_ref, xc_ref, bc_ref, z_ref, a_ref, dvec_ref, y_ref)

    @pl.when(d == 1)
    def _():
        _scan_core(True, de_ref, xc_ref, bc_ref, z_ref, a_ref, dvec_ref, y_ref)


def _outproj_body(yf_ref, yb_ref, wf_ref, wb_ref, x_ref, o_ref):
    o_ref[...] = (x_ref[...]
                  + jnp.dot(yf_ref[...].astype(jnp.bfloat16), wf_ref[...],
                            preferred_element_type=jnp.float32)
                  + jnp.dot(yb_ref[...].astype(jnp.bfloat16), wb_ref[...],
                            preferred_element_type=jnp.float32))


def kernel(x, gamma, beta,
           f_in_w, f_conv_w, f_conv_b, f_xproj_w, f_dt_w, f_dt_b, f_A_log, f_D, f_out_w,
           b_in_w, b_conv_w, b_conv_b, b_xproj_w, b_dt_w, b_dt_b, b_A_log, b_D, b_out_w):
    f32 = jnp.float32
    xr = x.reshape(BA, L // 8, 8, DM).transpose(0, 2, 1, 3).reshape(BA * L, DM)
    w_in = jnp.concatenate([f_in_w, b_in_w], axis=0).T.astype(jnp.bfloat16)
    cw_f = f_conv_w[:, ::-1].T
    cw_b = b_conv_w[:, ::-1].T
    cb_f = f_conv_b.reshape(1, 1, DI)
    cb_b = b_conv_b.reshape(1, 1, DI)
    a_f = -jnp.exp(f_A_log.T).reshape(1, NS, DI)
    a_b = -jnp.exp(b_A_log.T).reshape(1, NS, DI)
    d_f = f_D.reshape(1, 1, DI)
    d_b = b_D.reshape(1, 1, DI)
    w_out_f = f_out_w.T.astype(jnp.bfloat16)
    w_out_b = b_out_w.T.astype(jnp.bfloat16)

    xz = pl.pallas_call(
        _ln_inproj_body,
        grid=(4, 8),
        in_specs=[
            pl.BlockSpec((256, DM), lambda j, i: (i, 0)),
            pl.BlockSpec((1, DM), lambda j, i: (0, 0)),
            pl.BlockSpec((1, DM), lambda j, i: (0, 0)),
            pl.BlockSpec((DM, DI), lambda j, i: (0, j)),
        ],
        out_specs=pl.BlockSpec((256, DI), lambda j, i: (i, j)),
        out_shape=jax.ShapeDtypeStruct((BA * L, 4 * DI), f32),
        compiler_params=pltpu.CompilerParams(
            dimension_semantics=("parallel", "arbitrary"),
            vmem_limit_bytes=48 * 1024 * 1024),
        name="ln_inproj",
    )(xr, gamma.reshape(1, DM), beta.reshape(1, DM), w_in)
    xz3 = xz.reshape(BA, L, 4 * DI)

    def conv_call(reverse):
        dirn = 1 if reverse else 0
        cw = cw_b if reverse else cw_f
        cb = cb_b if reverse else cb_f
        xp = b_xproj_w if reverse else f_xproj_w
        dtw = b_dt_w if reverse else f_dt_w
        dtb = b_dt_b if reverse else f_dt_b
        return pl.pallas_call(
            functools.partial(_conv_core, reverse),
            grid=(BA,),
            in_specs=[
                pl.BlockSpec((1, L, DI), lambda b, d=dirn: (b, 0, 2 * d)),
                pl.BlockSpec((1, 4, DI), lambda b: (0, 0, 0)),
                pl.BlockSpec((1, 1, DI), lambda b: (0, 0, 0)),
                pl.BlockSpec((1, DI, RK + 2 * NS), lambda b: (0, 0, 0)),
                pl.BlockSpec((1, RK, DI), lambda b: (0, 0, 0)),
                pl.BlockSpec((1, 1, DI), lambda b: (0, 0, 0)),
            ],
            out_specs=[
                pl.BlockSpec((1, L, DI), lambda b: (b, 0, 0)),
                pl.BlockSpec((1, L, DI), lambda b: (b, 0, 0)),
                pl.BlockSpec((1, L, 2 * NS), lambda b: (b, 0, 0)),
            ],
            out_shape=[
                jax.ShapeDtypeStruct((BA, L, DI), f32),
                jax.ShapeDtypeStruct((BA, L, DI), f32),
                jax.ShapeDtypeStruct((BA, L, 2 * NS), f32),
            ],
            compiler_params=pltpu.CompilerParams(
                dimension_semantics=("parallel",),
                vmem_limit_bytes=52 * 1024 * 1024),
            name="conv_bwd" if reverse else "conv_fwd",
        )(xz3, cw.reshape(1, 4, DI), cb, xp.T.reshape(1, DI, RK + 2 * NS),
          dtw.T.reshape(1, RK, DI), dtb.reshape(1, 1, DI))

    de_f, xc_f, bc_f = conv_call(False)
    de_b, xc_b, bc_b = conv_call(True)

    def scan_call(reverse, de_a, xc_a, bc_a):
        dirn = 1 if reverse else 0
        zoff = dirn * 2 * NT + NT
        av = a_b if reverse else a_f
        dvec = d_b if reverse else d_f
        return pl.pallas_call(
            functools.partial(_scan_core, reverse),
            grid=(BA, NT),
            in_specs=[
                pl.BlockSpec((1, L, DT), lambda b, j: (b, 0, j)),
                pl.BlockSpec((1, L, DT), lambda b, j: (b, 0, j)),
                pl.BlockSpec((1, L, 2 * NS), lambda b, j: (b, 0, 0)),
                pl.BlockSpec((1, L, DT), lambda b, j, zo=zoff: (b, 0, zo + j)),
                pl.BlockSpec((1, NS, DT), lambda b, j: (0, 0, j)),
                pl.BlockSpec((1, 1, DT), lambda b, j: (0, 0, j)),
            ],
            out_specs=pl.BlockSpec((1, L, DT), lambda b, j: (b, 0, j)),
            out_shape=jax.ShapeDtypeStruct((BA, L, DI), f32),
            compiler_params=pltpu.CompilerParams(
                dimension_semantics=("parallel", "arbitrary"),
                vmem_limit_bytes=48 * 1024 * 1024),
            name="scan_bwd" if reverse else "scan_fwd",
        )(de_a, xc_a, bc_a, xz3, av, dvec)

    yg_f = scan_call(False, de_f, xc_f, bc_f).reshape(BA * L, DI)
    yg_b = scan_call(True, de_b, xc_b, bc_b).reshape(BA * L, DI)

    out = pl.pallas_call(
        _outproj_body,
        grid=(8,),
        in_specs=[
            pl.BlockSpec((256, DI), lambda i: (i, 0)),
            pl.BlockSpec((256, DI), lambda i: (i, 0)),
            pl.BlockSpec((DI, DM), lambda i: (0, 0)),
            pl.BlockSpec((DI, DM), lambda i: (0, 0)),
            pl.BlockSpec((256, DM), lambda i: (i, 0)),
        ],
        out_specs=pl.BlockSpec((256, DM), lambda i: (i, 0)),
        out_shape=jax.ShapeDtypeStruct((BA * L, DM), f32),
        compiler_params=pltpu.CompilerParams(
            dimension_semantics=("parallel",),
            vmem_limit_bytes=48 * 1024 * 1024),
        name="outproj",
    )(yg_f, yg_b, w_out_f, w_out_b, xr)
    return (out.reshape(BA, 8, L // 8, DM).transpose(0, 2, 1, 3)
            .reshape(BA, L, DM))
